# baseline (speedup 1.0000x reference)
import functools

import jax
import jax.numpy as jnp
from jax import lax
from jax.experimental import pallas as pl
from jax.experimental.pallas import tpu as pltpu
from jax.experimental.pallas import tpu_sc as plsc

B, N, F = 1024, 128, 128
NC, NS, L = 2, 16, 16
NW = NC * NS
BPW = B // NW
FC = F // L

def _i16():
    return lax.iota(jnp.int32, 16)


def _full16(v):
    return jnp.full((L,), v, dtype=jnp.int32)


def _sc_body(x_hbm, nodes_hbm, adj2d_hbm, w2d_hbm, nn_hbm,
             nodes_out_hbm, agg_hbm, nnp1_hbm,
             nn_v, idx_v, nnp1_v, xrows_v, arows_v, wrows_v, agg_v,
             buf0, buf1, sem_g, sem_in0, sem_in1, sem_out0, sem_out1):
    wid = lax.axis_index("s") * NC + lax.axis_index("c")
    base = wid * BPW
    bufs = [buf0, buf1]
    sem_in = [sem_in0, sem_in1]
    sem_out = [sem_out0, sem_out1]

    descs_in = [None] * BPW
    descs_out = [None] * BPW
    descs_in[0] = pltpu.async_copy(nodes_hbm.at[base], buf0, sem_in0)

    pltpu.sync_copy(nn_hbm.at[pl.ds(base, BPW)], nn_v)
    pltpu.sync_copy(x_hbm.at[pl.ds(base, BPW)], xrows_v)
    for c in range(BPW // L):
        nnc = nn_v[pl.ds(c * L, L)]
        idx_v[pl.ds(c * L, L)] = (base + c * L + _i16()) * N + nnc
        nnp1_v[pl.ds(c * L, L)] = nnc + 1
    pltpu.sync_copy(nnp1_v, nnp1_hbm.at[pl.ds(base, BPW)])

    pltpu.async_copy(adj2d_hbm.at[idx_v], arows_v, sem_g).wait()
    pltpu.async_copy(w2d_hbm.at[idx_v], wrows_v, sem_g).wait()

    for k in range(BPW):
        s = k % 2
        buf = bufs[s]
        descs_in[k].wait()

        if k + 1 < BPW:
            if k - 1 >= 0:
                descs_out[k - 1].wait()
            s2 = (k + 1) % 2
            descs_in[k + 1] = pltpu.async_copy(
                nodes_hbm.at[base + k + 1], bufs[s2], sem_in[s2])

        r = nn_v[pl.ds((k // L) * L, L)][k % L]
        for c in range(FC):
            buf[r, pl.ds(c * L, L)] = xrows_v[k, pl.ds(c * L, L)]

        awcs = [arows_v[k, pl.ds(cj * L, L)] * wrows_v[k, pl.ds(cj * L, L)]
                for cj in range(FC)]

        def j2body(j2, acc):
            accl = list(acc)
            idxv = jnp.full((L,), j2, jnp.int32)
            for cj in range(FC):
                awb = jnp.take_along_axis(awcs[cj], idxv, axis=0,
                                          mode="promise_in_bounds")
                row = cj * L + j2
                for c in range(FC):
                    accl[c] = accl[c] + awb * buf[row, pl.ds(c * L, L)]
            return tuple(accl)

        acc = lax.fori_loop(0, L, j2body,
                            tuple(jnp.zeros((L,), jnp.float32)
                                  for _ in range(FC)))
        for c in range(FC):
            agg_v[k, pl.ds(c * L, L)] = acc[c]

        descs_out[k] = pltpu.async_copy(
            buf, nodes_out_hbm.at[base + k], sem_out[s])

    descs_out[BPW - 2].wait()
    descs_out[BPW - 1].wait()
    pltpu.sync_copy(agg_v, agg_hbm.at[pl.ds(base, BPW)])


_sc_step = functools.partial(
    pl.kernel,
    out_type=(
        jax.ShapeDtypeStruct((B, N, F), jnp.float32),
        jax.ShapeDtypeStruct((B, F), jnp.float32),
        jax.ShapeDtypeStruct((B,), jnp.int32),
    ),
    mesh=plsc.VectorSubcoreMesh(core_axis_name="c", subcore_axis_name="s",
                                num_cores=NC, num_subcores=NS),
    compiler_params=pltpu.CompilerParams(needs_layout_passes=False),
    scratch_types=[
        pltpu.VMEM((BPW,), jnp.int32),
        pltpu.VMEM((BPW,), jnp.int32),
        pltpu.VMEM((BPW,), jnp.int32),
        pltpu.VMEM((BPW, F), jnp.float32),
        pltpu.VMEM((BPW, N), jnp.float32),
        pltpu.VMEM((BPW, N), jnp.float32),
        pltpu.VMEM((BPW, F), jnp.float32),
        pltpu.VMEM((N, F), jnp.float32),
        pltpu.VMEM((N, F), jnp.float32),
        pltpu.SemaphoreType.DMA,
        pltpu.SemaphoreType.DMA,
        pltpu.SemaphoreType.DMA,
        pltpu.SemaphoreType.DMA,
        pltpu.SemaphoreType.DMA,
    ],
)(_sc_body)


_NCHUNK = 8


def _copy_body(a_ref, b_ref, ao_ref, bo_ref, sems):
    cs = B // _NCHUNK
    descs = []
    for i in range(_NCHUNK):
        sl = pl.ds(i * cs, cs)
        descs.append(pltpu.make_async_copy(a_ref.at[sl], ao_ref.at[sl],
                                           sems.at[0, i]))
        descs.append(pltpu.make_async_copy(b_ref.at[sl], bo_ref.at[sl],
                                           sems.at[1, i]))
    for d in descs:
        d.start()
    for d in descs:
        d.wait()


_tc_copy = pl.pallas_call(
    _copy_body,
    in_specs=[
        pl.BlockSpec(memory_space=pl.ANY),
        pl.BlockSpec(memory_space=pl.ANY),
    ],
    out_specs=[
        pl.BlockSpec(memory_space=pl.ANY),
        pl.BlockSpec(memory_space=pl.ANY),
    ],
    scratch_shapes=[pltpu.SemaphoreType.DMA((2, _NCHUNK))],
    out_shape=(
        jax.ShapeDtypeStruct((B, N, N), jnp.float32),
        jax.ShapeDtypeStruct((B, N, N), jnp.float32),
    ),
)


def _tc_body(agg_ref, w_ref, mx_ref):
    mx_ref[...] = jnp.tanh(
        jnp.dot(agg_ref[...], w_ref[...],
                preferred_element_type=jnp.float32,
                precision=lax.Precision.HIGHEST))


_tc_finish = pl.pallas_call(
    _tc_body,
    grid=(B // 128,),
    in_specs=[
        pl.BlockSpec((128, F), lambda i: (i, 0)),
        pl.BlockSpec((F, F), lambda i: (0, 0)),
    ],
    out_specs=pl.BlockSpec((128, F), lambda i: (i, 0)),
    out_shape=jax.ShapeDtypeStruct((B, F), jnp.float32),
)


def kernel(x, nodes, adj, weights, num_nodes, W):
    nn = num_nodes.astype(jnp.int32)
    adj2d = adj.reshape(B * N, N)
    w2d = weights.reshape(B * N, N)
    nodes_out, agg, nnp1 = _sc_step(x, nodes, adj2d, w2d, nn)
    adj_out, w_out = _tc_copy(adj, weights)
    mx = _tc_finish(agg, W)
    return (mx, nodes_out, adj_out, w_out, nnp1.astype(num_nodes.dtype))


# speedup vs baseline: 26.9008x; 26.9008x over previous
import functools

import jax
import jax.numpy as jnp
from jax import lax
from jax.experimental import pallas as pl
from jax.experimental.pallas import tpu as pltpu
from jax.experimental.pallas import tpu_sc as plsc

B, N, F = 1024, 128, 128
NC, NS, L = 2, 16, 16
NW = NC * NS
BPW = B // NW
FC = F // L

def _i16():
    return lax.iota(jnp.int32, 16)


def _full16(v):
    return jnp.full((L,), v, dtype=jnp.int32)


def _sc_body(x_hbm, nodes_hbm, adj2d_hbm, w2d_hbm, nn_hbm,
             nodes_out_hbm, agg_hbm, nnp1_hbm,
             nn_v, idx_v, nnp1_v, xrows_v, arows_v, wrows_v, agg_v,
             buf0, buf1, sem_g, sem_in0, sem_in1, sem_out0, sem_out1):
    wid = lax.axis_index("s") * NC + lax.axis_index("c")
    base = wid * BPW
    bufs = [buf0, buf1]
    sem_in = [sem_in0, sem_in1]
    sem_out = [sem_out0, sem_out1]

    descs_in = [None] * BPW
    descs_out = [None] * BPW
    descs_in[0] = pltpu.async_copy(nodes_hbm.at[base], buf0, sem_in0)

    pltpu.sync_copy(nn_hbm.at[pl.ds(base, BPW)], nn_v)
    pltpu.sync_copy(x_hbm.at[pl.ds(base, BPW)], xrows_v)
    for c in range(BPW // L):
        nnc = nn_v[pl.ds(c * L, L)]
        idx_v[pl.ds(c * L, L)] = (base + c * L + _i16()) * N + nnc
        nnp1_v[pl.ds(c * L, L)] = nnc + 1
    pltpu.sync_copy(nnp1_v, nnp1_hbm.at[pl.ds(base, BPW)])

    pltpu.async_copy(adj2d_hbm.at[idx_v], arows_v, sem_g).wait()
    pltpu.async_copy(w2d_hbm.at[idx_v], wrows_v, sem_g).wait()

    for k in range(BPW):
        s = k % 2
        buf = bufs[s]
        descs_in[k].wait()

        if k + 1 < BPW:
            if k - 1 >= 0:
                descs_out[k - 1].wait()
            s2 = (k + 1) % 2
            descs_in[k + 1] = pltpu.async_copy(
                nodes_hbm.at[base + k + 1], bufs[s2], sem_in[s2])

        r = nn_v[pl.ds((k // L) * L, L)][k % L]
        for c in range(FC):
            buf[r, pl.ds(c * L, L)] = xrows_v[k, pl.ds(c * L, L)]

        awcs = [arows_v[k, pl.ds(cj * L, L)] * wrows_v[k, pl.ds(cj * L, L)]
                for cj in range(FC)]

        def j2body(j2, acc):
            accl = list(acc)
            idxv = jnp.full((L,), j2, jnp.int32)
            for cj in range(FC):
                awb = jnp.take_along_axis(awcs[cj], idxv, axis=0,
                                          mode="promise_in_bounds")
                row = cj * L + j2
                for c in range(FC):
                    accl[c] = accl[c] + awb * buf[row, pl.ds(c * L, L)]
            return tuple(accl)

        acc = lax.fori_loop(0, L, j2body,
                            tuple(jnp.zeros((L,), jnp.float32)
                                  for _ in range(FC)))
        for c in range(FC):
            agg_v[k, pl.ds(c * L, L)] = acc[c]

        descs_out[k] = pltpu.async_copy(
            buf, nodes_out_hbm.at[base + k], sem_out[s])

    descs_out[BPW - 2].wait()
    descs_out[BPW - 1].wait()
    pltpu.sync_copy(agg_v, agg_hbm.at[pl.ds(base, BPW)])


_sc_step = functools.partial(
    pl.kernel,
    out_type=(
        jax.ShapeDtypeStruct((B, N, F), jnp.float32),
        jax.ShapeDtypeStruct((B, F), jnp.float32),
        jax.ShapeDtypeStruct((B,), jnp.int32),
    ),
    mesh=plsc.VectorSubcoreMesh(core_axis_name="c", subcore_axis_name="s",
                                num_cores=NC, num_subcores=NS),
    compiler_params=pltpu.CompilerParams(needs_layout_passes=False),
    scratch_types=[
        pltpu.VMEM((BPW,), jnp.int32),
        pltpu.VMEM((BPW,), jnp.int32),
        pltpu.VMEM((BPW,), jnp.int32),
        pltpu.VMEM((BPW, F), jnp.float32),
        pltpu.VMEM((BPW, N), jnp.float32),
        pltpu.VMEM((BPW, N), jnp.float32),
        pltpu.VMEM((BPW, F), jnp.float32),
        pltpu.VMEM((N, F), jnp.float32),
        pltpu.VMEM((N, F), jnp.float32),
        pltpu.SemaphoreType.DMA,
        pltpu.SemaphoreType.DMA,
        pltpu.SemaphoreType.DMA,
        pltpu.SemaphoreType.DMA,
        pltpu.SemaphoreType.DMA,
    ],
)(_sc_body)


_CB = 32


def _copy_body(a_ref, b_ref, ao_ref, bo_ref):
    ao_ref[...] = a_ref[...]
    bo_ref[...] = b_ref[...]


_tc_copy = pl.pallas_call(
    _copy_body,
    grid=(B // _CB,),
    in_specs=[
        pl.BlockSpec((_CB, N, N), lambda i: (i, 0, 0)),
        pl.BlockSpec((_CB, N, N), lambda i: (i, 0, 0)),
    ],
    out_specs=[
        pl.BlockSpec((_CB, N, N), lambda i: (i, 0, 0)),
        pl.BlockSpec((_CB, N, N), lambda i: (i, 0, 0)),
    ],
    out_shape=(
        jax.ShapeDtypeStruct((B, N, N), jnp.float32),
        jax.ShapeDtypeStruct((B, N, N), jnp.float32),
    ),
)


def _tc_body(agg_ref, w_ref, mx_ref):
    mx_ref[...] = jnp.tanh(
        jnp.dot(agg_ref[...], w_ref[...],
                preferred_element_type=jnp.float32,
                precision=lax.Precision.HIGHEST))


_tc_finish = pl.pallas_call(
    _tc_body,
    grid=(B // 128,),
    in_specs=[
        pl.BlockSpec((128, F), lambda i: (i, 0)),
        pl.BlockSpec((F, F), lambda i: (0, 0)),
    ],
    out_specs=pl.BlockSpec((128, F), lambda i: (i, 0)),
    out_shape=jax.ShapeDtypeStruct((B, F), jnp.float32),
)


def kernel(x, nodes, adj, weights, num_nodes, W):
    nn = num_nodes.astype(jnp.int32)
    adj2d = adj.reshape(B * N, N)
    w2d = weights.reshape(B * N, N)
    nodes_out, agg, nnp1 = _sc_step(x, nodes, adj2d, w2d, nn)
    adj_out, w_out = _tc_copy(adj, weights)
    mx = _tc_finish(agg, W)
    return (mx, nodes_out, adj_out, w_out, nnp1.astype(num_nodes.dtype))


# speedup vs baseline: 27.6278x; 1.0270x over previous
import functools

import jax
import jax.numpy as jnp
from jax import lax
from jax.experimental import pallas as pl
from jax.experimental.pallas import tpu as pltpu
from jax.experimental.pallas import tpu_sc as plsc

B, N, F = 1024, 128, 128
NC, NS, L = 2, 16, 16
NW = NC * NS
BPW = B // NW
FC = F // L

def _i16():
    return lax.iota(jnp.int32, 16)


def _full16(v):
    return jnp.full((L,), v, dtype=jnp.int32)


def _sc_body(x_hbm, nodes_hbm, adj2d_hbm, w2d_hbm, nn_hbm,
             nodes_out_hbm, agg_hbm, nnp1_hbm,
             nn_v, idx_v, nnp1_v, xrows_v, arows_v, wrows_v, agg_v,
             buf0, buf1, sem_g, sem_in0, sem_in1, sem_out0, sem_out1):
    wid = lax.axis_index("s") * NC + lax.axis_index("c")
    base = wid * BPW
    bufs = [buf0, buf1]
    sem_in = [sem_in0, sem_in1]
    sem_out = [sem_out0, sem_out1]

    descs_in = [None] * BPW
    descs_out = [None] * BPW
    descs_in[0] = pltpu.async_copy(nodes_hbm.at[base], buf0, sem_in0)

    pltpu.sync_copy(nn_hbm.at[pl.ds(base, BPW)], nn_v)
    pltpu.sync_copy(x_hbm.at[pl.ds(base, BPW)], xrows_v)
    for c in range(BPW // L):
        nnc = nn_v[pl.ds(c * L, L)]
        idx_v[pl.ds(c * L, L)] = (base + c * L + _i16()) * N + nnc
        nnp1_v[pl.ds(c * L, L)] = nnc + 1
    pltpu.sync_copy(nnp1_v, nnp1_hbm.at[pl.ds(base, BPW)])

    pltpu.async_copy(adj2d_hbm.at[idx_v], arows_v, sem_g).wait()
    pltpu.async_copy(w2d_hbm.at[idx_v], wrows_v, sem_g).wait()

    for k in range(BPW):
        s = k % 2
        buf = bufs[s]
        descs_in[k].wait()

        if k + 1 < BPW:
            if k - 1 >= 0:
                descs_out[k - 1].wait()
            s2 = (k + 1) % 2
            descs_in[k + 1] = pltpu.async_copy(
                nodes_hbm.at[base + k + 1], bufs[s2], sem_in[s2])

        r = nn_v[pl.ds((k // L) * L, L)][k % L]
        for c in range(FC):
            buf[r, pl.ds(c * L, L)] = xrows_v[k, pl.ds(c * L, L)]

        awcs = [arows_v[k, pl.ds(cj * L, L)] * wrows_v[k, pl.ds(cj * L, L)]
                for cj in range(FC)]

        def j2body(j2, acc):
            accl = list(acc)
            idxv = jnp.full((L,), j2, jnp.int32)
            for cj in range(FC):
                awb = jnp.take_along_axis(awcs[cj], idxv, axis=0,
                                          mode="promise_in_bounds")
                row = cj * L + j2
                for c in range(FC):
                    accl[c] = accl[c] + awb * buf[row, pl.ds(c * L, L)]
            return tuple(accl)

        acc = lax.fori_loop(0, L, j2body,
                            tuple(jnp.zeros((L,), jnp.float32)
                                  for _ in range(FC)))
        for c in range(FC):
            agg_v[k, pl.ds(c * L, L)] = acc[c]

        descs_out[k] = pltpu.async_copy(
            buf, nodes_out_hbm.at[base + k], sem_out[s])

    descs_out[BPW - 2].wait()
    descs_out[BPW - 1].wait()
    pltpu.sync_copy(agg_v, agg_hbm.at[pl.ds(base, BPW)])


_sc_step = functools.partial(
    pl.kernel,
    out_type=(
        jax.ShapeDtypeStruct((B, N, F), jnp.float32),
        jax.ShapeDtypeStruct((B, F), jnp.float32),
        jax.ShapeDtypeStruct((B,), jnp.int32),
    ),
    mesh=plsc.VectorSubcoreMesh(core_axis_name="c", subcore_axis_name="s",
                                num_cores=NC, num_subcores=NS),
    compiler_params=pltpu.CompilerParams(needs_layout_passes=False),
    scratch_types=[
        pltpu.VMEM((BPW,), jnp.int32),
        pltpu.VMEM((BPW,), jnp.int32),
        pltpu.VMEM((BPW,), jnp.int32),
        pltpu.VMEM((BPW, F), jnp.float32),
        pltpu.VMEM((BPW, N), jnp.float32),
        pltpu.VMEM((BPW, N), jnp.float32),
        pltpu.VMEM((BPW, F), jnp.float32),
        pltpu.VMEM((N, F), jnp.float32),
        pltpu.VMEM((N, F), jnp.float32),
        pltpu.SemaphoreType.DMA,
        pltpu.SemaphoreType.DMA,
        pltpu.SemaphoreType.DMA,
        pltpu.SemaphoreType.DMA,
        pltpu.SemaphoreType.DMA,
    ],
)(_sc_body)


_CB = 64


def _copy_body(a_ref, b_ref, ao_ref, bo_ref):
    ao_ref[...] = a_ref[...]
    bo_ref[...] = b_ref[...]


_tc_copy = pl.pallas_call(
    _copy_body,
    grid=(B // _CB,),
    in_specs=[
        pl.BlockSpec((_CB, N, N), lambda i: (i, 0, 0)),
        pl.BlockSpec((_CB, N, N), lambda i: (i, 0, 0)),
    ],
    out_specs=[
        pl.BlockSpec((_CB, N, N), lambda i: (i, 0, 0)),
        pl.BlockSpec((_CB, N, N), lambda i: (i, 0, 0)),
    ],
    out_shape=(
        jax.ShapeDtypeStruct((B, N, N), jnp.float32),
        jax.ShapeDtypeStruct((B, N, N), jnp.float32),
    ),
)


def _tc_body(agg_ref, w_ref, mx_ref):
    mx_ref[...] = jnp.tanh(
        jnp.dot(agg_ref[...], w_ref[...],
                preferred_element_type=jnp.float32,
                precision=lax.Precision.HIGHEST))


_tc_finish = pl.pallas_call(
    _tc_body,
    out_shape=jax.ShapeDtypeStruct((B, F), jnp.float32),
)


def kernel(x, nodes, adj, weights, num_nodes, W):
    nn = num_nodes.astype(jnp.int32)
    adj2d = adj.reshape(B * N, N)
    w2d = weights.reshape(B * N, N)
    nodes_out, agg, nnp1 = _sc_step(x, nodes, adj2d, w2d, nn)
    adj_out, w_out = _tc_copy(adj, weights)
    mx = _tc_finish(agg, W)
    return (mx, nodes_out, adj_out, w_out, nnp1.astype(num_nodes.dtype))
